# Initial kernel scaffold; baseline (speedup 1.0000x reference)
import jax
import jax.numpy as jnp
from jax.experimental import pallas as pl


def kernel(x, edges, W1, b1, W2, b2, Wp, bp):
    raise NotImplementedError("write your pallas kernel here")


# speedup vs baseline: 7.1193x; 7.1193x over previous
import functools

import jax
import jax.numpy as jnp
from jax import lax
from jax.experimental import pallas as pl
from jax.experimental.pallas import tpu as pltpu
from jax.experimental.pallas import tpu_sc as plsc

N = 10000
E = 320000
D = 128

NC = 2
NS = 16
NW = NC * NS
CH = 128
E_PAD = 327680
CPW = E_PAD // (NW * CH)
QC = 16
N_PAD = 10240
RPT = N_PAD // NS

_mesh = plsc.VectorSubcoreMesh(core_axis_name="c", subcore_axis_name="s")


def _sc_body_common(sid, cid, rows, agg_sh, fill_val):
  fill16 = jnp.full((16,), fill_val, jnp.float32)
  zero16 = jnp.zeros((16,), jnp.float32)

  def fill(i, carry):
    for k in range(D // 16):
      rows[i, pl.ds(k * 16, 16)] = fill16
    return carry

  lax.fori_loop(0, CH, fill, 0)

  if fill_val != 0.0:
    def fill0(i, carry):
      for k in range(D // 16):
        rows[i, pl.ds(k * 16, 16)] = zero16
      return carry
    lax.fori_loop(0, CH, fill0, 0)
    for r in range(RPT // CH):
      row0 = sid * RPT + r * CH
      pltpu.sync_copy(rows, agg_sh.at[pl.ds(row0, CH)])
    lax.fori_loop(0, CH, fill, 0)
  else:
    for r in range(RPT // CH):
      row0 = sid * RPT + r * CH
      pltpu.sync_copy(rows, agg_sh.at[pl.ds(row0, CH)])


def _make_sc_agg():
  scratch = [
      pltpu.VMEM((QC, CH), jnp.int32),
      pltpu.VMEM((QC, CH), jnp.int32),
      pltpu.VMEM((CH, D), jnp.float32),
      pltpu.VMEM_SHARED((N_PAD, D), jnp.float32),
      pltpu.SemaphoreType.DMA,
  ]

  def body(x_hbm, src_hbm, dst_hbm, agg_out, srcv, dstv, rows, agg_sh, sem):
    cid = lax.axis_index("c")
    sid = lax.axis_index("s")
    wid = sid * NC + cid

    _sc_body_common(sid, cid, rows, agg_sh, 0.0)
    plsc.subcore_barrier()

    def quarter(q, carry):
      pltpu.sync_copy(src_hbm.at[pl.ds(wid * CPW + q * QC, QC)], srcv)
      pltpu.sync_copy(dst_hbm.at[pl.ds(wid * CPW + q * QC, QC)], dstv)

      def chunk(j, carry2):
        pltpu.async_copy(x_hbm.at[srcv.at[j]], rows, sem).wait()
        pltpu.sync_copy(rows, agg_sh.at[dstv.at[j]], add=True)
        return carry2

      lax.fori_loop(0, QC, chunk, 0)
      return carry

    lax.fori_loop(0, CPW // QC, quarter, 0)
    plsc.subcore_barrier()

    for r in range(RPT // CH):
      row0 = sid * RPT + r * CH
      pltpu.sync_copy(agg_sh.at[pl.ds(row0, CH)],
                      agg_out.at[pl.ds(cid * N_PAD + row0, CH)])

  return pl.kernel(
      body,
      out_type=jax.ShapeDtypeStruct((NC * N_PAD, D), jnp.float32),
      mesh=_mesh, scratch_types=scratch)


def _make_sc_cnt():
  scratch = [
      pltpu.VMEM((QC, CH), jnp.int32),
      pltpu.VMEM((CH, D), jnp.float32),
      pltpu.VMEM_SHARED((N_PAD, D), jnp.float32),
  ]

  def body(dst_hbm, cnt_out, dstv, rows, agg_sh):
    cid = lax.axis_index("c")
    sid = lax.axis_index("s")
    wid = sid * NC + cid

    _sc_body_common(sid, cid, rows, agg_sh, 1.0)
    plsc.subcore_barrier()

    def quarter(q, carry):
      pltpu.sync_copy(dst_hbm.at[pl.ds(wid * CPW + q * QC, QC)], dstv)

      def chunk(j, carry2):
        pltpu.sync_copy(rows, agg_sh.at[dstv.at[j]], add=True)
        return carry2

      lax.fori_loop(0, QC, chunk, 0)
      return carry

    lax.fori_loop(0, CPW // QC, quarter, 0)
    plsc.subcore_barrier()

    for r in range(RPT // CH):
      row0 = sid * RPT + r * CH
      pltpu.sync_copy(agg_sh.at[pl.ds(row0, CH)],
                      cnt_out.at[pl.ds(cid * N_PAD + row0, CH)])

  return pl.kernel(
      body,
      out_type=jax.ShapeDtypeStruct((NC * N_PAD, D), jnp.float32),
      mesh=_mesh, scratch_types=scratch)


_sc_agg = _make_sc_agg()
_sc_cnt = _make_sc_cnt()

BR = 1280


def _tc_layer_body(x_ref, agg_ref, cnt_ref, w_ref, b_ref, o_ref):
  cnt = cnt_ref[0, :, 0] + cnt_ref[1, :, 0]
  inv = 1.0 / jnp.maximum(cnt, 1.0)
  agg = agg_ref[0] + agg_ref[1]
  comb = x_ref[...] + agg * inv[:, None]
  h = lax.dot_general(comb, w_ref[...], (((1,), (1,)), ((), ())),
                      preferred_element_type=jnp.float32)
  o_ref[...] = jnp.maximum(h + b_ref[...], 0.0)


def _tc_final_body(x_ref, agg_ref, cnt_ref, w_ref, b_ref, wp_ref, bp_ref,
                   o_ref):
  cnt = cnt_ref[0, :, 0] + cnt_ref[1, :, 0]
  inv = 1.0 / jnp.maximum(cnt, 1.0)
  agg = agg_ref[0] + agg_ref[1]
  comb = x_ref[...] + agg * inv[:, None]
  h = lax.dot_general(comb, w_ref[...], (((1,), (1,)), ((), ())),
                      preferred_element_type=jnp.float32)
  h = jnp.maximum(h + b_ref[...], 0.0)
  p = lax.dot_general(h, wp_ref[...], (((1,), (1,)), ((), ())),
                      preferred_element_type=jnp.float32)
  o_ref[...] = p + bp_ref[...]


_row_spec = pl.BlockSpec((BR, D), lambda i: (i, 0))
_agg_spec = pl.BlockSpec((NC, BR, D), lambda i: (0, i, 0))
_w_spec = pl.BlockSpec((D, D), lambda i: (0, 0))
_b_spec = pl.BlockSpec((1, D), lambda i: (0, 0))

_tc_layer = pl.pallas_call(
    _tc_layer_body,
    grid=(N_PAD // BR,),
    in_specs=[_row_spec, _agg_spec, _agg_spec, _w_spec, _b_spec],
    out_specs=_row_spec,
    out_shape=jax.ShapeDtypeStruct((N_PAD, D), jnp.float32),
)

_tc_final = pl.pallas_call(
    _tc_final_body,
    grid=(N_PAD // BR,),
    in_specs=[_row_spec, _agg_spec, _agg_spec, _w_spec, _b_spec, _w_spec,
              _b_spec],
    out_specs=_row_spec,
    out_shape=jax.ShapeDtypeStruct((N_PAD, D), jnp.float32),
)


def kernel(x, edges, W1, b1, W2, b2, Wp, bp):
  src = edges[0]
  dst = edges[1]
  pad_idx = N + (jnp.arange(E_PAD - E, dtype=jnp.int32) % (N_PAD - N))
  src_p = jnp.concatenate([src, pad_idx]).reshape(NW * CPW, CH)
  dst_p = jnp.concatenate([dst, pad_idx]).reshape(NW * CPW, CH)
  x_pad = jnp.zeros((N_PAD, D), jnp.float32).at[:N].set(x)

  cnt1 = _sc_cnt(dst_p).reshape(NC, N_PAD, D)
  agg1 = _sc_agg(x_pad, src_p, dst_p).reshape(NC, N_PAD, D)
  h1 = _tc_layer(x_pad, agg1, cnt1, W1, b1.reshape(1, D))
  agg2 = _sc_agg(h1, src_p, dst_p).reshape(NC, N_PAD, D)
  pred = _tc_final(h1, agg2, cnt1, W2, b2.reshape(1, D), Wp,
                   bp.reshape(1, D))
  return pred[:N]


# speedup vs baseline: 8.6731x; 1.2183x over previous
import functools

import jax
import jax.numpy as jnp
from jax import lax
from jax.experimental import pallas as pl
from jax.experimental.pallas import tpu as pltpu
from jax.experimental.pallas import tpu_sc as plsc

N = 10000
E = 320000
D = 128

NC = 2
NS = 16
NW = NC * NS
CH = 128
E_PAD = 327680
CPW = E_PAD // (NW * CH)
QC = 16
N_PAD = 10240
RPT = N_PAD // NS

_mesh = plsc.VectorSubcoreMesh(core_axis_name="c", subcore_axis_name="s")


def _sc_body_common(sid, cid, rows, agg_sh, fill_val):
  fill16 = jnp.full((16,), fill_val, jnp.float32)
  zero16 = jnp.zeros((16,), jnp.float32)

  def fill(i, carry):
    for k in range(D // 16):
      rows[i, pl.ds(k * 16, 16)] = fill16
    return carry

  lax.fori_loop(0, CH, fill, 0)

  if fill_val != 0.0:
    def fill0(i, carry):
      for k in range(D // 16):
        rows[i, pl.ds(k * 16, 16)] = zero16
      return carry
    lax.fori_loop(0, CH, fill0, 0)
    for r in range(RPT // CH):
      row0 = sid * RPT + r * CH
      pltpu.sync_copy(rows, agg_sh.at[pl.ds(row0, CH)])
    lax.fori_loop(0, CH, fill, 0)
  else:
    for r in range(RPT // CH):
      row0 = sid * RPT + r * CH
      pltpu.sync_copy(rows, agg_sh.at[pl.ds(row0, CH)])


def _make_sc_agg():
  scratch = [
      pltpu.VMEM((QC, CH), jnp.int32),
      pltpu.VMEM((QC, CH), jnp.int32),
      pltpu.VMEM((CH, D), jnp.float32),
      pltpu.VMEM((CH, D), jnp.float32),
      pltpu.VMEM_SHARED((N_PAD, D), jnp.float32),
      pltpu.SemaphoreType.DMA,
      pltpu.SemaphoreType.DMA,
  ]

  def body(x_hbm, src_hbm, dst_hbm, agg_out, srcv, dstv, rows0, rows1,
           agg_sh, sem0, sem1):
    cid = lax.axis_index("c")
    sid = lax.axis_index("s")
    wid = sid * NC + cid

    _sc_body_common(sid, cid, rows0, agg_sh, 0.0)
    plsc.subcore_barrier()

    bufs = (rows0, rows1)
    sems = (sem0, sem1)

    def quarter(q, carry):
      pltpu.sync_copy(src_hbm.at[pl.ds(wid * CPW + q * QC, QC)], srcv)
      pltpu.sync_copy(dst_hbm.at[pl.ds(wid * CPW + q * QC, QC)], dstv)

      pltpu.async_copy(x_hbm.at[srcv.at[0]], rows0, sem0)
      for j in range(QC):
        b = j % 2
        pltpu.make_async_copy(x_hbm.at[srcv.at[j]], bufs[b],
                              sems[b]).wait()
        if j + 1 < QC:
          pltpu.async_copy(x_hbm.at[srcv.at[j + 1]], bufs[1 - b],
                           sems[1 - b])
        pltpu.sync_copy(bufs[b], agg_sh.at[dstv.at[j]], add=True)
      return carry

    lax.fori_loop(0, CPW // QC, quarter, 0)
    plsc.subcore_barrier()

    for r in range(RPT // CH):
      row0 = sid * RPT + r * CH
      pltpu.sync_copy(agg_sh.at[pl.ds(row0, CH)],
                      agg_out.at[pl.ds(cid * N_PAD + row0, CH)])

  return pl.kernel(
      body,
      out_type=jax.ShapeDtypeStruct((NC * N_PAD, D), jnp.float32),
      mesh=_mesh, scratch_types=scratch)


def _make_sc_cnt():
  scratch = [
      pltpu.VMEM((QC, CH), jnp.int32),
      pltpu.VMEM((CH, D), jnp.float32),
      pltpu.VMEM_SHARED((N_PAD, D), jnp.float32),
  ]

  def body(dst_hbm, cnt_out, dstv, rows, agg_sh):
    cid = lax.axis_index("c")
    sid = lax.axis_index("s")
    wid = sid * NC + cid

    _sc_body_common(sid, cid, rows, agg_sh, 1.0)
    plsc.subcore_barrier()

    def quarter(q, carry):
      pltpu.sync_copy(dst_hbm.at[pl.ds(wid * CPW + q * QC, QC)], dstv)

      def chunk(j, carry2):
        pltpu.sync_copy(rows, agg_sh.at[dstv.at[j]], add=True)
        return carry2

      lax.fori_loop(0, QC, chunk, 0)
      return carry

    lax.fori_loop(0, CPW // QC, quarter, 0)
    plsc.subcore_barrier()

    for r in range(RPT // CH):
      row0 = sid * RPT + r * CH
      pltpu.sync_copy(agg_sh.at[pl.ds(row0, CH)],
                      cnt_out.at[pl.ds(cid * N_PAD + row0, CH)])

  return pl.kernel(
      body,
      out_type=jax.ShapeDtypeStruct((NC * N_PAD, D), jnp.float32),
      mesh=_mesh, scratch_types=scratch)


_sc_agg = _make_sc_agg()
_sc_cnt = _make_sc_cnt()

BR = 1280


def _tc_layer_body(x_ref, agg_ref, cnt_ref, w_ref, b_ref, o_ref):
  cnt = cnt_ref[0, :, 0] + cnt_ref[1, :, 0]
  inv = 1.0 / jnp.maximum(cnt, 1.0)
  agg = agg_ref[0] + agg_ref[1]
  comb = x_ref[...] + agg * inv[:, None]
  h = lax.dot_general(comb, w_ref[...], (((1,), (1,)), ((), ())),
                      preferred_element_type=jnp.float32)
  o_ref[...] = jnp.maximum(h + b_ref[...], 0.0)


def _tc_final_body(x_ref, agg_ref, cnt_ref, w_ref, b_ref, wp_ref, bp_ref,
                   o_ref):
  cnt = cnt_ref[0, :, 0] + cnt_ref[1, :, 0]
  inv = 1.0 / jnp.maximum(cnt, 1.0)
  agg = agg_ref[0] + agg_ref[1]
  comb = x_ref[...] + agg * inv[:, None]
  h = lax.dot_general(comb, w_ref[...], (((1,), (1,)), ((), ())),
                      preferred_element_type=jnp.float32)
  h = jnp.maximum(h + b_ref[...], 0.0)
  p = lax.dot_general(h, wp_ref[...], (((1,), (1,)), ((), ())),
                      preferred_element_type=jnp.float32)
  o_ref[...] = p + bp_ref[...]


_row_spec = pl.BlockSpec((BR, D), lambda i: (i, 0))
_agg_spec = pl.BlockSpec((NC, BR, D), lambda i: (0, i, 0))
_w_spec = pl.BlockSpec((D, D), lambda i: (0, 0))
_b_spec = pl.BlockSpec((1, D), lambda i: (0, 0))

_tc_layer = pl.pallas_call(
    _tc_layer_body,
    grid=(N_PAD // BR,),
    in_specs=[_row_spec, _agg_spec, _agg_spec, _w_spec, _b_spec],
    out_specs=_row_spec,
    out_shape=jax.ShapeDtypeStruct((N_PAD, D), jnp.float32),
)

_tc_final = pl.pallas_call(
    _tc_final_body,
    grid=(N_PAD // BR,),
    in_specs=[_row_spec, _agg_spec, _agg_spec, _w_spec, _b_spec, _w_spec,
              _b_spec],
    out_specs=_row_spec,
    out_shape=jax.ShapeDtypeStruct((N_PAD, D), jnp.float32),
)


def kernel(x, edges, W1, b1, W2, b2, Wp, bp):
  src = edges[0]
  dst = edges[1]
  pad_idx = N + (jnp.arange(E_PAD - E, dtype=jnp.int32) % (N_PAD - N))
  src_p = jnp.concatenate([src, pad_idx]).reshape(NW * CPW, CH)
  dst_p = jnp.concatenate([dst, pad_idx]).reshape(NW * CPW, CH)
  x_pad = jnp.zeros((N_PAD, D), jnp.float32).at[:N].set(x)

  cnt1 = _sc_cnt(dst_p).reshape(NC, N_PAD, D)
  agg1 = _sc_agg(x_pad, src_p, dst_p).reshape(NC, N_PAD, D)
  h1 = _tc_layer(x_pad, agg1, cnt1, W1, b1.reshape(1, D))
  agg2 = _sc_agg(h1, src_p, dst_p).reshape(NC, N_PAD, D)
  pred = _tc_final(h1, agg2, cnt1, W2, b2.reshape(1, D), Wp,
                   bp.reshape(1, D))
  return pred[:N]


# speedup vs baseline: 8.8621x; 1.0218x over previous
import functools

import jax
import jax.numpy as jnp
from jax import lax
from jax.experimental import pallas as pl
from jax.experimental.pallas import tpu as pltpu
from jax.experimental.pallas import tpu_sc as plsc

N = 10000
E = 320000
D = 128

NC = 2
NS = 16
NW = NC * NS
CH = 128
E_PAD = 327680
CPW = E_PAD // (NW * CH)
QC = 16
N_PAD = 10240
RPT = N_PAD // NS

_mesh = plsc.VectorSubcoreMesh(core_axis_name="c", subcore_axis_name="s")


def _sc_body_common(sid, cid, rows, agg_sh, fill_val):
  fill16 = jnp.full((16,), fill_val, jnp.float32)
  zero16 = jnp.zeros((16,), jnp.float32)

  def fill(i, carry):
    for k in range(D // 16):
      rows[i, pl.ds(k * 16, 16)] = fill16
    return carry

  lax.fori_loop(0, CH, fill, 0)

  if fill_val != 0.0:
    def fill0(i, carry):
      for k in range(D // 16):
        rows[i, pl.ds(k * 16, 16)] = zero16
      return carry
    lax.fori_loop(0, CH, fill0, 0)
    for r in range(RPT // CH):
      row0 = sid * RPT + r * CH
      pltpu.sync_copy(rows, agg_sh.at[pl.ds(row0, CH)])
    lax.fori_loop(0, CH, fill, 0)
  else:
    for r in range(RPT // CH):
      row0 = sid * RPT + r * CH
      pltpu.sync_copy(rows, agg_sh.at[pl.ds(row0, CH)])


def _make_sc_agg():
  scratch = [
      pltpu.VMEM((QC, CH), jnp.int32),
      pltpu.VMEM((QC, CH), jnp.int32),
      pltpu.VMEM((CH, D), jnp.float32),
      pltpu.VMEM((CH, D), jnp.float32),
      pltpu.VMEM_SHARED((N_PAD, D), jnp.float32),
      pltpu.SemaphoreType.DMA,
      pltpu.SemaphoreType.DMA,
      pltpu.SemaphoreType.DMA,
      pltpu.SemaphoreType.DMA,
  ]

  def body(x_hbm, src_hbm, dst_hbm, agg_out, srcv, dstv, rows0, rows1,
           agg_sh, gsem0, gsem1, ssem0, ssem1):
    cid = lax.axis_index("c")
    sid = lax.axis_index("s")
    wid = sid * NC + cid

    _sc_body_common(sid, cid, rows0, agg_sh, 0.0)
    plsc.subcore_barrier()

    bufs = (rows0, rows1)
    gsems = (gsem0, gsem1)
    ssems = (ssem0, ssem1)

    def quarter(q, carry):
      pltpu.sync_copy(src_hbm.at[pl.ds(wid * CPW + q * QC, QC)], srcv)
      pltpu.sync_copy(dst_hbm.at[pl.ds(wid * CPW + q * QC, QC)], dstv)

      pltpu.async_copy(x_hbm.at[srcv.at[0]], rows0, gsem0)
      for j in range(QC):
        b = j % 2
        pltpu.make_async_copy(x_hbm.at[srcv.at[j]], bufs[b],
                              gsems[b]).wait()
        pltpu.async_copy(bufs[b], agg_sh.at[dstv.at[j]], ssems[b],
                         add=True)
        if j + 1 < QC:
          if j >= 1:
            pltpu.make_async_copy(bufs[1 - b],
                                  agg_sh.at[dstv.at[j - 1]],
                                  ssems[1 - b]).wait()
          pltpu.async_copy(x_hbm.at[srcv.at[j + 1]], bufs[1 - b],
                           gsems[1 - b])
      pltpu.make_async_copy(bufs[(QC - 2) % 2],
                            agg_sh.at[dstv.at[QC - 2]],
                            ssems[(QC - 2) % 2]).wait()
      pltpu.make_async_copy(bufs[(QC - 1) % 2],
                            agg_sh.at[dstv.at[QC - 1]],
                            ssems[(QC - 1) % 2]).wait()
      return carry

    lax.fori_loop(0, CPW // QC, quarter, 0)
    plsc.subcore_barrier()

    for r in range(RPT // CH):
      row0 = sid * RPT + r * CH
      pltpu.sync_copy(agg_sh.at[pl.ds(row0, CH)],
                      agg_out.at[pl.ds(cid * N_PAD + row0, CH)])

  return pl.kernel(
      body,
      out_type=jax.ShapeDtypeStruct((NC * N_PAD, D), jnp.float32),
      mesh=_mesh, scratch_types=scratch)


def _make_sc_cnt():
  scratch = [
      pltpu.VMEM((QC, CH), jnp.int32),
      pltpu.VMEM((CH, D), jnp.float32),
      pltpu.VMEM_SHARED((N_PAD, D), jnp.float32),
      pltpu.SemaphoreType.DMA,
  ]

  def body(dst_hbm, cnt_out, dstv, rows, agg_sh, ssem):
    cid = lax.axis_index("c")
    sid = lax.axis_index("s")
    wid = sid * NC + cid

    _sc_body_common(sid, cid, rows, agg_sh, 1.0)
    plsc.subcore_barrier()

    def quarter(q, carry):
      pltpu.sync_copy(dst_hbm.at[pl.ds(wid * CPW + q * QC, QC)], dstv)

      for j in range(QC):
        pltpu.async_copy(rows, agg_sh.at[dstv.at[j]], ssem, add=True)
      for j in range(QC):
        pltpu.make_async_copy(rows, agg_sh.at[dstv.at[j]], ssem).wait()
      return carry

    lax.fori_loop(0, CPW // QC, quarter, 0)
    plsc.subcore_barrier()

    for r in range(RPT // CH):
      row0 = sid * RPT + r * CH
      pltpu.sync_copy(agg_sh.at[pl.ds(row0, CH)],
                      cnt_out.at[pl.ds(cid * N_PAD + row0, CH)])

  return pl.kernel(
      body,
      out_type=jax.ShapeDtypeStruct((NC * N_PAD, D), jnp.float32),
      mesh=_mesh, scratch_types=scratch)


_sc_agg = _make_sc_agg()
_sc_cnt = _make_sc_cnt()

BR = 1280


def _tc_layer_body(x_ref, agg_ref, cnt_ref, w_ref, b_ref, o_ref):
  cnt = cnt_ref[0, :, 0] + cnt_ref[1, :, 0]
  inv = 1.0 / jnp.maximum(cnt, 1.0)
  agg = agg_ref[0] + agg_ref[1]
  comb = x_ref[...] + agg * inv[:, None]
  h = lax.dot_general(comb, w_ref[...], (((1,), (1,)), ((), ())),
                      preferred_element_type=jnp.float32)
  o_ref[...] = jnp.maximum(h + b_ref[...], 0.0)


def _tc_final_body(x_ref, agg_ref, cnt_ref, w_ref, b_ref, wp_ref, bp_ref,
                   o_ref):
  cnt = cnt_ref[0, :, 0] + cnt_ref[1, :, 0]
  inv = 1.0 / jnp.maximum(cnt, 1.0)
  agg = agg_ref[0] + agg_ref[1]
  comb = x_ref[...] + agg * inv[:, None]
  h = lax.dot_general(comb, w_ref[...], (((1,), (1,)), ((), ())),
                      preferred_element_type=jnp.float32)
  h = jnp.maximum(h + b_ref[...], 0.0)
  p = lax.dot_general(h, wp_ref[...], (((1,), (1,)), ((), ())),
                      preferred_element_type=jnp.float32)
  o_ref[...] = p + bp_ref[...]


_row_spec = pl.BlockSpec((BR, D), lambda i: (i, 0))
_agg_spec = pl.BlockSpec((NC, BR, D), lambda i: (0, i, 0))
_w_spec = pl.BlockSpec((D, D), lambda i: (0, 0))
_b_spec = pl.BlockSpec((1, D), lambda i: (0, 0))

_tc_layer = pl.pallas_call(
    _tc_layer_body,
    grid=(N_PAD // BR,),
    in_specs=[_row_spec, _agg_spec, _agg_spec, _w_spec, _b_spec],
    out_specs=_row_spec,
    out_shape=jax.ShapeDtypeStruct((N_PAD, D), jnp.float32),
)

_tc_final = pl.pallas_call(
    _tc_final_body,
    grid=(N_PAD // BR,),
    in_specs=[_row_spec, _agg_spec, _agg_spec, _w_spec, _b_spec, _w_spec,
              _b_spec],
    out_specs=_row_spec,
    out_shape=jax.ShapeDtypeStruct((N_PAD, D), jnp.float32),
)


def kernel(x, edges, W1, b1, W2, b2, Wp, bp):
  src = edges[0]
  dst = edges[1]
  pad_idx = N + (jnp.arange(E_PAD - E, dtype=jnp.int32) % (N_PAD - N))
  src_p = jnp.concatenate([src, pad_idx]).reshape(NW * CPW, CH)
  dst_p = jnp.concatenate([dst, pad_idx]).reshape(NW * CPW, CH)
  x_pad = jnp.zeros((N_PAD, D), jnp.float32).at[:N].set(x)

  cnt1 = _sc_cnt(dst_p).reshape(NC, N_PAD, D)
  agg1 = _sc_agg(x_pad, src_p, dst_p).reshape(NC, N_PAD, D)
  h1 = _tc_layer(x_pad, agg1, cnt1, W1, b1.reshape(1, D))
  agg2 = _sc_agg(h1, src_p, dst_p).reshape(NC, N_PAD, D)
  pred = _tc_final(h1, agg2, cnt1, W2, b2.reshape(1, D), Wp,
                   bp.reshape(1, D))
  return pred[:N]
